# Initial kernel scaffold; baseline (speedup 1.0000x reference)
import jax
import jax.numpy as jnp
from jax.experimental import pallas as pl


def kernel(x, edge_index, edge_attr, globalFeats, isTrain, W_rel1, b_rel1, W_root1, W_rel2, b_rel2, W_root2, Wg1, bg1, Wg2, bg2, Wg3, bg3, Wo1, bo1, Wo2, bo2):
    raise NotImplementedError("write your pallas kernel here")


# speedup vs baseline: 18.7562x; 18.7562x over previous
import functools

import jax
import jax.numpy as jnp
from jax import lax
from jax.experimental import pallas as pl
from jax.experimental.pallas import tpu as pltpu
from jax.experimental.pallas import tpu_sc as plsc

_B = 1024
_NPG = 54
_NP = 64
_EPG = 864
_D_IN = 64
_D_H = 128
_D_O = 4
_GLOB = 32

_G_CONV = 16
_G_HEAD = 256


def _build_adj(src, dst, w):
    info = plsc.get_sparse_core_info()
    n_workers = info.num_cores * info.num_subcores
    gpw = _B // n_workers
    mesh = plsc.VectorSubcoreMesh(core_axis_name="c", subcore_axis_name="s")

    @functools.partial(
        pl.kernel,
        out_type=jax.ShapeDtypeStruct((_B, _NP * _NP), jnp.float32),
        mesh=mesh,
        scratch_types=[
            pltpu.VMEM((_EPG,), jnp.int32),
            pltpu.VMEM((_EPG,), jnp.int32),
            pltpu.VMEM((_EPG,), jnp.float32),
            pltpu.VMEM((_NP * _NP,), jnp.float32),
        ],
        compiler_params=pltpu.CompilerParams(needs_layout_passes=False),
    )
    def build(src_hbm, dst_hbm, w_hbm, a_hbm, src_v, dst_v, w_v, acc_v):
        wid = lax.axis_index("s") * info.num_cores + lax.axis_index("c")

        def per_graph(gi, carry):
            g = wid * gpw + gi
            pltpu.sync_copy(src_hbm.at[pl.ds(g * _EPG, _EPG)], src_v)
            pltpu.sync_copy(dst_hbm.at[pl.ds(g * _EPG, _EPG)], dst_v)
            pltpu.sync_copy(w_hbm.at[pl.ds(g * _EPG, _EPG)], w_v)

            def zero16(j, c):
                acc_v[pl.ds(j * 16, 16)] = jnp.zeros((16,), jnp.float32)
                return c

            lax.fori_loop(0, _NP * _NP // 16, zero16, 0)

            goff = g * _NPG

            def edge16(i, c):
                s = src_v[pl.ds(i * 16, 16)]
                d = dst_v[pl.ds(i * 16, 16)]
                ww = w_v[pl.ds(i * 16, 16)]
                idx = (d - goff) * _NP + (s - goff)
                plsc.addupdate_scatter(acc_v, [idx], ww)
                return c

            lax.fori_loop(0, _EPG // 16, edge16, 0)
            pltpu.sync_copy(acc_v, a_hbm.at[g])
            return carry

        lax.fori_loop(0, gpw, per_graph, 0)

    return build(src, dst, w)


def _conv_body(a_ref, x_ref, wr1_ref, br1_ref, wq1_ref, wr2_ref, br2_ref,
               wq2_ref, out_ref):
    wr1 = wr1_ref[...]
    br1 = br1_ref[...]
    wq1 = wq1_ref[...]
    wr2 = wr2_ref[...]
    br2 = br2_ref[...]
    wq2 = wq2_ref[...]
    row_ok = lax.broadcasted_iota(jnp.int32, (_NP, _D_O), 0) < _NPG
    for r in range(_G_CONV):
        a = a_ref[r]
        xg = x_ref[r]
        aggr1 = jnp.dot(a, xg, preferred_element_type=jnp.float32)
        h1 = jnp.maximum(
            jnp.dot(aggr1, wr1, preferred_element_type=jnp.float32) + br1
            + jnp.dot(xg, wq1, preferred_element_type=jnp.float32), 0.0)
        aggr2 = jnp.dot(a, h1, preferred_element_type=jnp.float32)
        h2 = jnp.maximum(
            jnp.dot(aggr2, wr2, preferred_element_type=jnp.float32) + br2
            + jnp.dot(h1, wq2, preferred_element_type=jnp.float32), 0.0)
        out_ref[r] = jnp.where(row_ok, h2, 0.0)


def _conv(a, x_pad, w_rel1, b_rel1, w_root1, w_rel2, b_rel2, w_root2,
          interpret=False):
    a3 = a.reshape(_B, _NP, _NP)
    grid = (_B // _G_CONV,)
    blk = lambda shape: pl.BlockSpec(shape, lambda i: (i, 0, 0))
    full2 = lambda shape: pl.BlockSpec(shape, lambda i: (0, 0))
    return pl.pallas_call(
        _conv_body,
        grid=grid,
        in_specs=[
            blk((_G_CONV, _NP, _NP)),
            blk((_G_CONV, _NP, _D_IN)),
            full2((_D_IN, _D_H)),
            full2((1, _D_H)),
            full2((_D_IN, _D_H)),
            full2((_D_H, _D_O)),
            full2((1, _D_O)),
            full2((_D_H, _D_O)),
        ],
        out_specs=blk((_G_CONV, _NP, _D_O)),
        out_shape=jax.ShapeDtypeStruct((_B, _NP, _D_O), jnp.float32),
        interpret=interpret,
    )(a3, x_pad, w_rel1, b_rel1.reshape(1, _D_H), w_root1,
      w_rel2, b_rel2.reshape(1, _D_O), w_root2)


def _head_body(e_ref, gf_ref, wg1_ref, bg1_ref, wg2_ref, bg2_ref, wg3_ref,
               bg3_ref, w1e_ref, w1g_ref, bo1_ref, wo2_ref, bo2_ref, out_ref):
    gv = gf_ref[...]
    gv = jnp.maximum(jnp.dot(gv, wg1_ref[...],
                             preferred_element_type=jnp.float32)
                     + bg1_ref[...], 0.0)
    gv = jnp.maximum(jnp.dot(gv, wg2_ref[...],
                             preferred_element_type=jnp.float32)
                     + bg2_ref[...], 0.0)
    gv = jnp.maximum(jnp.dot(gv, wg3_ref[...],
                             preferred_element_type=jnp.float32)
                     + bg3_ref[...], 0.0)
    z = (jnp.dot(e_ref[...], w1e_ref[...], preferred_element_type=jnp.float32)
         + jnp.dot(gv, w1g_ref[...], preferred_element_type=jnp.float32)
         + bo1_ref[...])
    z = jnp.maximum(z, 0.0)
    z = jnp.dot(z, wo2_ref[...], preferred_element_type=jnp.float32) \
        + bo2_ref[...]
    out_ref[...] = jax.nn.sigmoid(z)


def _head(emb, gf, wg1, bg1, wg2, bg2, wg3, bg3, w1e, w1g, bo1, wo2, bo2,
          interpret=False):
    grid = (_B // _G_HEAD,)
    row = lambda shape: pl.BlockSpec(shape, lambda i: (i, 0))
    full = lambda shape: pl.BlockSpec(shape, lambda i: (0, 0))
    return pl.pallas_call(
        _head_body,
        grid=grid,
        in_specs=[
            row((_G_HEAD, _NP * _D_O)),
            row((_G_HEAD, _GLOB)),
            full((_GLOB, 8)),
            full((1, 8)),
            full((8, 8)),
            full((1, 8)),
            full((8, _GLOB)),
            full((1, _GLOB)),
            full((_NP * _D_O, 128)),
            full((_GLOB, 128)),
            full((1, 128)),
            full((128, 1)),
            full((1, 1)),
        ],
        out_specs=row((_G_HEAD, 1)),
        out_shape=jax.ShapeDtypeStruct((_B, 1), jnp.float32),
        interpret=interpret,
    )(emb, gf, wg1, bg1.reshape(1, 8), wg2, bg2.reshape(1, 8), wg3,
      bg3.reshape(1, _GLOB), w1e, w1g, bo1.reshape(1, 128), wo2,
      bo2.reshape(1, 1))


def kernel(x, edge_index, edge_attr, globalFeats, isTrain, W_rel1, b_rel1,
           W_root1, W_rel2, b_rel2, W_root2, Wg1, bg1, Wg2, bg2, Wg3, bg3,
           Wo1, bo1, Wo2, bo2):
    a = _build_adj(edge_index[0], edge_index[1], edge_attr)

    x_pad = jnp.pad(x.reshape(_B, _NPG, _D_IN),
                    ((0, 0), (0, _NP - _NPG), (0, 0)))

    h2 = _conv(a, x_pad, W_rel1, b_rel1, W_root1, W_rel2, b_rel2, W_root2)
    emb = h2.reshape(_B, _NP * _D_O)

    w1e = jnp.concatenate(
        [Wo1[:_NPG * _D_O], jnp.zeros(((_NP - _NPG) * _D_O, 128),
                                      jnp.float32)], axis=0)
    w1g = Wo1[_NPG * _D_O:]

    return _head(emb, globalFeats, Wg1, bg1, Wg2, bg2, Wg3, bg3,
                 w1e, w1g, bo1, Wo2, bo2)


# speedup vs baseline: 41.3760x; 2.2060x over previous
import functools

import jax
import jax.numpy as jnp
from jax import lax
from jax.experimental import pallas as pl
from jax.experimental.pallas import tpu as pltpu
from jax.experimental.pallas import tpu_sc as plsc

_B = 1024
_NPG = 54
_NP = 64
_EPG = 864
_D_IN = 64
_D_H = 128
_D_O = 4
_GLOB = 32

_G_CONV = 16
_G_HEAD = 256


def _build_adj(edge_index, w):
    info = plsc.get_sparse_core_info()
    n_workers = info.num_cores * info.num_subcores
    gpw = _B // n_workers
    epw = gpw * _EPG
    mesh = plsc.VectorSubcoreMesh(core_axis_name="c", subcore_axis_name="s")

    @functools.partial(
        pl.kernel,
        out_type=jax.ShapeDtypeStruct((_B, _NP * _NP), jnp.float32),
        mesh=mesh,
        scratch_types=[
            pltpu.VMEM((epw,), jnp.int32),
            pltpu.VMEM((epw,), jnp.int32),
            pltpu.VMEM((epw,), jnp.float32),
            pltpu.VMEM((_NP * _NP,), jnp.float32),
            pltpu.VMEM((_NP * _NP,), jnp.float32),
            pltpu.SemaphoreType.DMA,
            pltpu.SemaphoreType.DMA,
            pltpu.SemaphoreType.DMA,
            pltpu.SemaphoreType.DMA,
        ],
        compiler_params=pltpu.CompilerParams(needs_layout_passes=False),
    )
    def build(ei_hbm, w_hbm, a_hbm, src_v, dst_v, w_v, acc0, acc1,
              sem_s, sem_d, sem_w, sem_o):
        wid = lax.axis_index("s") * info.num_cores + lax.axis_index("c")
        ebase = wid * epw
        cs = pltpu.async_copy(ei_hbm.at[0, pl.ds(ebase, epw)], src_v, sem_s)
        cd = pltpu.async_copy(ei_hbm.at[1, pl.ds(ebase, epw)], dst_v, sem_d)
        cw = pltpu.async_copy(w_hbm.at[pl.ds(ebase, epw)], w_v, sem_w)
        cs.wait()
        cd.wait()
        cw.wait()

        accs = (acc0, acc1)
        pending = [None, None]
        for gi in range(gpw):
            acc = accs[gi % 2]
            if pending[gi % 2] is not None:
                pending[gi % 2].wait()

            def zero64(j, c, acc=acc):
                for u in range(4):
                    acc[pl.ds(j * 64 + u * 16, 16)] = jnp.zeros(
                        (16,), jnp.float32)
                return c

            lax.fori_loop(0, _NP * _NP // 64, zero64, 0)

            g = wid * gpw + gi
            goff = g * _NPG
            e0 = gi * _EPG

            def edge48(i, c, acc=acc, e0=e0, goff=goff):
                for u in range(3):
                    o = e0 + i * 48 + u * 16
                    s = src_v[pl.ds(o, 16)]
                    d = dst_v[pl.ds(o, 16)]
                    ww = w_v[pl.ds(o, 16)]
                    idx = (d - goff) * _NP + (s - goff)
                    plsc.addupdate_scatter(acc, [idx], ww)
                return c

            lax.fori_loop(0, _EPG // 48, edge48, 0)
            pending[gi % 2] = pltpu.async_copy(acc, a_hbm.at[g], sem_o)
        for p in pending:
            if p is not None:
                p.wait()

    return build(edge_index, w)


def _conv_body(a_ref, x_ref, wr1_ref, br1_ref, wq1_ref, wr2_ref, br2_ref,
               wq2_ref, out_ref, xp_s, ag1_s, h1_s, ag2_s):
    pad = jnp.zeros((_NP - _NPG, _D_IN), jnp.float32)
    for r in range(_G_CONV):
        xp_r = jnp.concatenate([x_ref[r], pad], axis=0)
        xp_s[pl.ds(r * _NP, _NP), :] = xp_r
        ag1_s[pl.ds(r * _NP, _NP), :] = jnp.dot(
            a_ref[r], xp_r, preferred_element_type=jnp.float32)
    x_all = xp_s[...]
    h1 = jnp.maximum(
        jnp.dot(ag1_s[...], wr1_ref[...], preferred_element_type=jnp.float32)
        + br1_ref[...]
        + jnp.dot(x_all, wq1_ref[...], preferred_element_type=jnp.float32),
        0.0)
    h1_s[...] = h1
    for r in range(_G_CONV):
        ag2_s[pl.ds(r * _NP, _NP), :] = jnp.dot(
            a_ref[r], h1_s[pl.ds(r * _NP, _NP), :],
            preferred_element_type=jnp.float32)
    h2 = jnp.maximum(
        jnp.dot(ag2_s[...], wr2_ref[...], preferred_element_type=jnp.float32)
        + br2_ref[...]
        + jnp.dot(h1_s[...], wq2_ref[...],
                  preferred_element_type=jnp.float32),
        0.0)
    row_ok = (lax.broadcasted_iota(jnp.int32, (_G_CONV * _NP, _D_O), 0)
              % _NP) < _NPG
    out_ref[...] = jnp.where(row_ok, h2, 0.0)


def _conv(a, x3, w_rel1, b_rel1, w_root1, w_rel2, b_rel2, w_root2,
          interpret=False):
    a3 = a.reshape(_B, _NP, _NP)
    grid = (_B // _G_CONV,)
    blk3 = lambda shape: pl.BlockSpec(shape, lambda i: (i, 0, 0))
    full2 = lambda shape: pl.BlockSpec(shape, lambda i: (0, 0))
    return pl.pallas_call(
        _conv_body,
        grid=grid,
        in_specs=[
            blk3((_G_CONV, _NP, _NP)),
            blk3((_G_CONV, _NPG, _D_IN)),
            full2((_D_IN, _D_H)),
            full2((1, _D_H)),
            full2((_D_IN, _D_H)),
            full2((_D_H, _D_O)),
            full2((1, _D_O)),
            full2((_D_H, _D_O)),
        ],
        out_specs=pl.BlockSpec((_G_CONV * _NP, _D_O), lambda i: (i, 0)),
        out_shape=jax.ShapeDtypeStruct((_B * _NP, _D_O), jnp.float32),
        scratch_shapes=[
            pltpu.VMEM((_G_CONV * _NP, _D_IN), jnp.float32),
            pltpu.VMEM((_G_CONV * _NP, _D_IN), jnp.float32),
            pltpu.VMEM((_G_CONV * _NP, _D_H), jnp.float32),
            pltpu.VMEM((_G_CONV * _NP, _D_H), jnp.float32),
        ],
        interpret=interpret,
    )(a3, x3, w_rel1, b_rel1.reshape(1, _D_H), w_root1,
      w_rel2, b_rel2.reshape(1, _D_O), w_root2)


def _head_body(e_ref, gf_ref, wg1_ref, bg1_ref, wg2_ref, bg2_ref, wg3_ref,
               bg3_ref, w1e_ref, w1g_ref, bo1_ref, wo2_ref, bo2_ref, out_ref):
    gv = gf_ref[...]
    gv = jnp.maximum(jnp.dot(gv, wg1_ref[...],
                             preferred_element_type=jnp.float32)
                     + bg1_ref[...], 0.0)
    gv = jnp.maximum(jnp.dot(gv, wg2_ref[...],
                             preferred_element_type=jnp.float32)
                     + bg2_ref[...], 0.0)
    gv = jnp.maximum(jnp.dot(gv, wg3_ref[...],
                             preferred_element_type=jnp.float32)
                     + bg3_ref[...], 0.0)
    z = (jnp.dot(e_ref[...], w1e_ref[...], preferred_element_type=jnp.float32)
         + jnp.dot(gv, w1g_ref[...], preferred_element_type=jnp.float32)
         + bo1_ref[...])
    z = jnp.maximum(z, 0.0)
    z = jnp.dot(z, wo2_ref[...], preferred_element_type=jnp.float32) \
        + bo2_ref[...]
    out_ref[...] = jax.nn.sigmoid(z)


def _head(emb, gf, wg1, bg1, wg2, bg2, wg3, bg3, w1e, w1g, bo1, wo2, bo2,
          interpret=False):
    grid = (_B // _G_HEAD,)
    row = lambda shape: pl.BlockSpec(shape, lambda i: (i, 0))
    full = lambda shape: pl.BlockSpec(shape, lambda i: (0, 0))
    return pl.pallas_call(
        _head_body,
        grid=grid,
        in_specs=[
            row((_G_HEAD, _NP * _D_O)),
            row((_G_HEAD, _GLOB)),
            full((_GLOB, 8)),
            full((1, 8)),
            full((8, 8)),
            full((1, 8)),
            full((8, _GLOB)),
            full((1, _GLOB)),
            full((_NP * _D_O, 128)),
            full((_GLOB, 128)),
            full((1, 128)),
            full((128, 1)),
            full((1, 1)),
        ],
        out_specs=row((_G_HEAD, 1)),
        out_shape=jax.ShapeDtypeStruct((_B, 1), jnp.float32),
        interpret=interpret,
    )(emb, gf, wg1, bg1.reshape(1, 8), wg2, bg2.reshape(1, 8), wg3,
      bg3.reshape(1, _GLOB), w1e, w1g, bo1.reshape(1, 128), wo2,
      bo2.reshape(1, 1))


def kernel(x, edge_index, edge_attr, globalFeats, isTrain, W_rel1, b_rel1,
           W_root1, W_rel2, b_rel2, W_root2, Wg1, bg1, Wg2, bg2, Wg3, bg3,
           Wo1, bo1, Wo2, bo2):
    a = _build_adj(edge_index, edge_attr)

    x3 = x.reshape(_B, _NPG, _D_IN)
    h2 = _conv(a, x3, W_rel1, b_rel1, W_root1, W_rel2, b_rel2, W_root2)
    emb = h2.reshape(_B, _NP * _D_O)

    w1e = jnp.concatenate(
        [Wo1[:_NPG * _D_O], jnp.zeros(((_NP - _NPG) * _D_O, 128),
                                      jnp.float32)], axis=0)
    w1g = Wo1[_NPG * _D_O:]

    return _head(emb, globalFeats, Wg1, bg1, Wg2, bg2, Wg3, bg3,
                 w1e, w1g, bo1, Wo2, bo2)


# speedup vs baseline: 68.6491x; 1.6592x over previous
import functools

import jax
import jax.numpy as jnp
from jax import lax
from jax.experimental import pallas as pl
from jax.experimental.pallas import tpu as pltpu
from jax.experimental.pallas import tpu_sc as plsc

_B = 1024
_NPG = 54
_NP = 64
_NR = 128
_EPG = 864
_D_IN = 64
_D_H = 128
_D_O = 4
_GLOB = 32

_G_CONV = 16


def _build_adj(edge_index, w):
    info = plsc.get_sparse_core_info()
    n_workers = info.num_cores * info.num_subcores
    gpw = _B // n_workers
    epw = gpw * _EPG
    mesh = plsc.VectorSubcoreMesh(core_axis_name="c", subcore_axis_name="s")

    @functools.partial(
        pl.kernel,
        out_type=jax.ShapeDtypeStruct((_B, _NP * _NR), jnp.float32),
        mesh=mesh,
        scratch_types=[
            pltpu.VMEM((epw,), jnp.int32),
            pltpu.VMEM((epw,), jnp.int32),
            pltpu.VMEM((epw,), jnp.float32),
            pltpu.VMEM((_NP * _NR,), jnp.float32),
            pltpu.VMEM((_NP * _NR,), jnp.float32),
            pltpu.SemaphoreType.DMA,
            pltpu.SemaphoreType.DMA,
            pltpu.SemaphoreType.DMA,
            pltpu.SemaphoreType.DMA,
        ],
        compiler_params=pltpu.CompilerParams(needs_layout_passes=False),
    )
    def build(ei_hbm, w_hbm, a_hbm, src_v, dst_v, w_v, acc0, acc1,
              sem_s, sem_d, sem_w, sem_o):
        wid = lax.axis_index("s") * info.num_cores + lax.axis_index("c")
        ebase = wid * epw
        cs = pltpu.async_copy(ei_hbm.at[0, pl.ds(ebase, epw)], src_v, sem_s)
        cd = pltpu.async_copy(ei_hbm.at[1, pl.ds(ebase, epw)], dst_v, sem_d)
        cw = pltpu.async_copy(w_hbm.at[pl.ds(ebase, epw)], w_v, sem_w)

        def zero_all(j, c):
            for u in range(4):
                acc0[pl.ds(j * 64 + u * 16, 16)] = jnp.zeros(
                    (16,), jnp.float32)
                acc1[pl.ds(j * 64 + u * 16, 16)] = jnp.zeros(
                    (16,), jnp.float32)
            return c

        lax.fori_loop(0, _NP * _NR // 64, zero_all, 0)
        cs.wait()
        cd.wait()
        cw.wait()

        accs = (acc0, acc1)
        pending = [None, None]
        for gi in range(gpw):
            acc = accs[gi % 2]
            if pending[gi % 2] is not None:
                pending[gi % 2].wait()

            if gi >= 2:
                def zero_rows(d, c, acc=acc):
                    for u in range(4):
                        acc[pl.ds(d * _NR + u * 16, 16)] = jnp.zeros(
                            (16,), jnp.float32)
                    return c

                lax.fori_loop(0, _NPG, zero_rows, 0)

            g = wid * gpw + gi
            goff = g * _NPG
            e0 = gi * _EPG

            def edge48(i, c, acc=acc, e0=e0, goff=goff):
                for u in range(3):
                    o = e0 + i * 48 + u * 16
                    s = src_v[pl.ds(o, 16)]
                    d = dst_v[pl.ds(o, 16)]
                    ww = w_v[pl.ds(o, 16)]
                    idx = (d - goff) * _NR + (s - goff)
                    plsc.addupdate_scatter(acc, [idx], ww)
                return c

            lax.fori_loop(0, _EPG // 48, edge48, 0)
            pending[gi % 2] = pltpu.async_copy(acc, a_hbm.at[g], sem_o)
        for p in pending:
            if p is not None:
                p.wait()

    return build(edge_index, w)


def _net_body(a_ref, x_ref, gf_ref, wr1_ref, br1_ref, wq1_ref, wr2_ref,
              br2_ref, wq2_ref, wg1_ref, bg1_ref, wg2_ref, bg2_ref, wg3_ref,
              bg3_ref, w2e_ref, w1g_ref, bo1_ref, wo2_ref, bo2_ref, out_ref,
              xp_s, ag1_s, h1_s, ag2_s):
    pad_src = jnp.zeros((_NR - _NPG, _D_IN), jnp.float32)
    for r in range(_G_CONV):
        x_r = x_ref[pl.ds(r * _NPG, _NPG), :]
        xp128 = jnp.concatenate([x_r, pad_src], axis=0)
        a_r = jnp.reshape(a_ref[r], (_NP, _NR))
        ag1_s[pl.ds(r * _NP, _NP), :] = jnp.dot(
            a_r, xp128, preferred_element_type=jnp.float32)
        xp_s[pl.ds(r * _NP, _NP), :] = xp128[:_NP]
    h1 = jnp.maximum(
        jnp.dot(ag1_s[...], wr1_ref[...], preferred_element_type=jnp.float32)
        + br1_ref[...]
        + jnp.dot(xp_s[...], wq1_ref[...],
                  preferred_element_type=jnp.float32),
        0.0)
    h1_s[...] = h1
    pad_h = jnp.zeros((_NR - _NP, _D_H), jnp.float32)
    for r in range(_G_CONV):
        a_r = jnp.reshape(a_ref[r], (_NP, _NR))
        h1p = jnp.concatenate([h1_s[pl.ds(r * _NP, _NP), :], pad_h], axis=0)
        ag2_s[pl.ds(r * _NP, _NP), :] = jnp.dot(
            a_r, h1p, preferred_element_type=jnp.float32)
    h2 = jnp.maximum(
        jnp.dot(ag2_s[...], wr2_ref[...], preferred_element_type=jnp.float32)
        + br2_ref[...]
        + jnp.dot(h1_s[...], wq2_ref[...],
                  preferred_element_type=jnp.float32),
        0.0)
    row_ok = (lax.broadcasted_iota(jnp.int32, (_G_CONV * _NP, _D_O), 0)
              % _NP) < _NPG
    h2 = jnp.where(row_ok, h2, 0.0)

    h2t = jnp.transpose(h2, (1, 0))
    m = jnp.reshape(h2t, (_D_O, _G_CONV // 2, 2 * _NP))
    z2 = jnp.zeros((_G_CONV // 2, 2 * _D_H), jnp.float32)
    for c in range(_D_O):
        z2 = z2 + jnp.dot(m[c], w2e_ref[c],
                          preferred_element_type=jnp.float32)
    ze = jnp.reshape(z2, (_G_CONV, _D_H))

    gv = gf_ref[...]
    gv = jnp.maximum(jnp.dot(gv, wg1_ref[...],
                             preferred_element_type=jnp.float32)
                     + bg1_ref[...], 0.0)
    gv = jnp.maximum(jnp.dot(gv, wg2_ref[...],
                             preferred_element_type=jnp.float32)
                     + bg2_ref[...], 0.0)
    gv = jnp.maximum(jnp.dot(gv, wg3_ref[...],
                             preferred_element_type=jnp.float32)
                     + bg3_ref[...], 0.0)

    z = jnp.maximum(
        ze + jnp.dot(gv, w1g_ref[...], preferred_element_type=jnp.float32)
        + bo1_ref[...], 0.0)
    z = jnp.dot(z, wo2_ref[...], preferred_element_type=jnp.float32) \
        + bo2_ref[...]
    out_ref[...] = jax.nn.sigmoid(z)


def _net(a, x, gf, w_rel1, b_rel1, w_root1, w_rel2, b_rel2, w_root2,
         wg1, bg1, wg2, bg2, wg3, bg3, w2e, w1g, bo1, wo2, bo2,
         interpret=False):
    grid = (_B // _G_CONV,)
    row = lambda shape: pl.BlockSpec(shape, lambda i: (i, 0))
    full2 = lambda shape: pl.BlockSpec(shape, lambda i: (0, 0))
    return pl.pallas_call(
        _net_body,
        grid=grid,
        in_specs=[
            row((_G_CONV, _NP * _NR)),
            row((_G_CONV * _NPG, _D_IN)),
            row((_G_CONV, _GLOB)),
            full2((_D_IN, _D_H)),
            full2((1, _D_H)),
            full2((_D_IN, _D_H)),
            full2((_D_H, _D_O)),
            full2((1, _D_O)),
            full2((_D_H, _D_O)),
            full2((_GLOB, 8)),
            full2((1, 8)),
            full2((8, 8)),
            full2((1, 8)),
            full2((8, _GLOB)),
            full2((1, _GLOB)),
            pl.BlockSpec((_D_O, _NR, 2 * _D_H), lambda i: (0, 0, 0)),
            full2((_GLOB, _D_H)),
            full2((1, _D_H)),
            full2((_D_H, 1)),
            full2((1, 1)),
        ],
        out_specs=row((_G_CONV, 1)),
        out_shape=jax.ShapeDtypeStruct((_B, 1), jnp.float32),
        scratch_shapes=[
            pltpu.VMEM((_G_CONV * _NP, _D_IN), jnp.float32),
            pltpu.VMEM((_G_CONV * _NP, _D_IN), jnp.float32),
            pltpu.VMEM((_G_CONV * _NP, _D_H), jnp.float32),
            pltpu.VMEM((_G_CONV * _NP, _D_H), jnp.float32),
        ],
        interpret=interpret,
    )(a, x, gf, w_rel1, b_rel1.reshape(1, _D_H), w_root1,
      w_rel2, b_rel2.reshape(1, _D_O), w_root2,
      wg1, bg1.reshape(1, 8), wg2, bg2.reshape(1, 8), wg3,
      bg3.reshape(1, _GLOB), w2e, w1g, bo1.reshape(1, _D_H), wo2,
      bo2.reshape(1, 1))


def _prep_head_weights(Wo1):
    w1r = Wo1[:_NPG * _D_O].reshape(_NPG, _D_O, _D_H)
    base = jnp.pad(w1r, ((0, _NP - _NPG), (0, 0), (0, 0)))
    base = base.transpose(1, 0, 2)
    zblk = jnp.zeros((_D_O, _NP, _D_H), jnp.float32)
    top = jnp.concatenate([base, zblk], axis=2)
    bot = jnp.concatenate([zblk, base], axis=2)
    return jnp.concatenate([top, bot], axis=1)


def kernel(x, edge_index, edge_attr, globalFeats, isTrain, W_rel1, b_rel1,
           W_root1, W_rel2, b_rel2, W_root2, Wg1, bg1, Wg2, bg2, Wg3, bg3,
           Wo1, bo1, Wo2, bo2):
    a = _build_adj(edge_index, edge_attr)
    w2e = _prep_head_weights(Wo1)
    w1g = Wo1[_NPG * _D_O:]
    return _net(a, x, globalFeats, W_rel1, b_rel1, W_root1,
                W_rel2, b_rel2, W_root2, Wg1, bg1, Wg2, bg2, Wg3, bg3,
                w2e, w1g, bo1, Wo2, bo2)


# speedup vs baseline: 92.0418x; 1.3408x over previous
import functools

import jax
import jax.numpy as jnp
from jax import lax
from jax.experimental import pallas as pl
from jax.experimental.pallas import tpu as pltpu
from jax.experimental.pallas import tpu_sc as plsc

_B = 1024
_NPG = 54
_NP = 64
_NR = 128
_EPG = 864
_D_IN = 64
_D_H = 128
_D_O = 4
_GLOB = 32

_G_CONV = 32


def _build_adj(edge_index, w):
    info = plsc.get_sparse_core_info()
    n_workers = info.num_cores * info.num_subcores
    gpw = _B // n_workers
    epw = gpw * _EPG
    mesh = plsc.VectorSubcoreMesh(core_axis_name="c", subcore_axis_name="s")

    @functools.partial(
        pl.kernel,
        out_type=jax.ShapeDtypeStruct((_B, _NP * _NR), jnp.float32),
        mesh=mesh,
        scratch_types=[
            pltpu.VMEM((epw,), jnp.int32),
            pltpu.VMEM((epw,), jnp.int32),
            pltpu.VMEM((epw,), jnp.float32),
            pltpu.VMEM((_NP * _NR,), jnp.float32),
            pltpu.VMEM((_NP * _NR,), jnp.float32),
            pltpu.SemaphoreType.DMA,
            pltpu.SemaphoreType.DMA,
            pltpu.SemaphoreType.DMA,
            pltpu.SemaphoreType.DMA,
        ],
        compiler_params=pltpu.CompilerParams(needs_layout_passes=False),
        cost_estimate=pl.CostEstimate(
            flops=2_000_000, bytes_accessed=45_000_000, transcendentals=0),
    )
    def build(ei_hbm, w_hbm, a_hbm, src_v, dst_v, w_v, acc0, acc1,
              sem_s, sem_d, sem_w, sem_o):
        wid = lax.axis_index("s") * info.num_cores + lax.axis_index("c")
        ebase = wid * epw
        cs = pltpu.async_copy(ei_hbm.at[0, pl.ds(ebase, epw)], src_v, sem_s)
        cd = pltpu.async_copy(ei_hbm.at[1, pl.ds(ebase, epw)], dst_v, sem_d)
        cw = pltpu.async_copy(w_hbm.at[pl.ds(ebase, epw)], w_v, sem_w)

        def zero_all(j, c):
            for u in range(4):
                acc0[pl.ds(j * 64 + u * 16, 16)] = jnp.zeros(
                    (16,), jnp.float32)
                acc1[pl.ds(j * 64 + u * 16, 16)] = jnp.zeros(
                    (16,), jnp.float32)
            return c

        lax.fori_loop(0, _NP * _NR // 64, zero_all, 0)
        cs.wait()
        cd.wait()
        cw.wait()

        accs = (acc0, acc1)
        pending = [None, None]
        for gi in range(gpw):
            acc = accs[gi % 2]
            if pending[gi % 2] is not None:
                pending[gi % 2].wait()

            if gi >= 2:
                def zero_rows(d, c, acc=acc):
                    for u in range(4):
                        acc[pl.ds(d * _NR + u * 16, 16)] = jnp.zeros(
                            (16,), jnp.float32)
                    return c

                lax.fori_loop(0, _NPG, zero_rows, 0)

            g = wid * gpw + gi
            goff = g * _NPG
            e0 = gi * _EPG

            def edge48(i, c, acc=acc, e0=e0, goff=goff):
                for u in range(3):
                    o = e0 + i * 48 + u * 16
                    s = src_v[pl.ds(o, 16)]
                    d = dst_v[pl.ds(o, 16)]
                    ww = w_v[pl.ds(o, 16)]
                    idx = (d - goff) * _NR + (s - goff)
                    plsc.addupdate_scatter(acc, [idx], ww)
                return c

            lax.fori_loop(0, _EPG // 48, edge48, 0)
            pending[gi % 2] = pltpu.async_copy(acc, a_hbm.at[g], sem_o)
        for p in pending:
            if p is not None:
                p.wait()

    return build(edge_index, w)


def _net_body(a_ref, x_ref, gf_ref, w1cat_ref, br1_ref, w2cat_ref,
              br2_ref, wg1_ref, bg1_ref, wg2_ref, bg2_ref, wg3_ref,
              bg3_ref, w2e_ref, w1g_ref, bo1_ref, wo2_ref, bo2_ref, out_ref,
              cat1_s, cat2_s):
    pad_src = jnp.zeros((_NR - _NPG, _D_IN), jnp.float32)
    for r in range(_G_CONV):
        x_r = x_ref[pl.ds(r * _NPG, _NPG), :]
        xp128 = jnp.concatenate([x_r, pad_src], axis=0)
        a_r = jnp.reshape(a_ref[r], (_NP, _NR))
        cat1_s[pl.ds(r * _NP, _NP), :_D_IN] = jnp.dot(
            a_r, xp128, preferred_element_type=jnp.float32)
        cat1_s[pl.ds(r * _NP, _NP), _D_IN:] = xp128[:_NP]
    h1 = jnp.maximum(
        jnp.dot(cat1_s[...], w1cat_ref[...],
                preferred_element_type=jnp.float32) + br1_ref[...],
        0.0)
    cat2_s[:, _D_H:] = h1
    pad_h = jnp.zeros((_NR - _NP, _D_H), jnp.float32)
    for r in range(_G_CONV):
        a_r = jnp.reshape(a_ref[r], (_NP, _NR))
        h1p = jnp.concatenate(
            [cat2_s[pl.ds(r * _NP, _NP), _D_H:], pad_h], axis=0)
        cat2_s[pl.ds(r * _NP, _NP), :_D_H] = jnp.dot(
            a_r, h1p, preferred_element_type=jnp.float32)
    h2 = jnp.maximum(
        jnp.dot(cat2_s[...], w2cat_ref[...],
                preferred_element_type=jnp.float32) + br2_ref[...],
        0.0)
    row_ok = (lax.broadcasted_iota(jnp.int32, (_G_CONV * _NP, _D_O), 0)
              % _NP) < _NPG
    h2 = jnp.where(row_ok, h2, 0.0)

    h2t = jnp.transpose(h2, (1, 0))
    m = jnp.reshape(h2t, (_D_O, _G_CONV // 2, 2 * _NP))
    z2 = jnp.zeros((_G_CONV // 2, 2 * _D_H), jnp.float32)
    for c in range(_D_O):
        z2 = z2 + jnp.dot(m[c], w2e_ref[c],
                          preferred_element_type=jnp.float32)
    ze = jnp.reshape(z2, (_G_CONV, _D_H))

    gv = gf_ref[...]
    gv = jnp.maximum(jnp.dot(gv, wg1_ref[...],
                             preferred_element_type=jnp.float32)
                     + bg1_ref[...], 0.0)
    gv = jnp.maximum(jnp.dot(gv, wg2_ref[...],
                             preferred_element_type=jnp.float32)
                     + bg2_ref[...], 0.0)
    gv = jnp.maximum(jnp.dot(gv, wg3_ref[...],
                             preferred_element_type=jnp.float32)
                     + bg3_ref[...], 0.0)

    z = jnp.maximum(
        ze + jnp.dot(gv, w1g_ref[...], preferred_element_type=jnp.float32)
        + bo1_ref[...], 0.0)
    z = jnp.dot(z, wo2_ref[...], preferred_element_type=jnp.float32) \
        + bo2_ref[...]
    out_ref[...] = jax.nn.sigmoid(z)


def _net(a, x, gf, w_rel1, b_rel1, w_root1, w_rel2, b_rel2, w_root2,
         wg1, bg1, wg2, bg2, wg3, bg3, w2e, w1g, bo1, wo2, bo2,
         interpret=False):
    grid = (_B // _G_CONV,)
    row = lambda shape: pl.BlockSpec(shape, lambda i: (i, 0))
    full2 = lambda shape: pl.BlockSpec(shape, lambda i: (0, 0))
    w1cat = jnp.concatenate([w_rel1, w_root1], axis=0)
    w2cat = jnp.concatenate([w_rel2, w_root2], axis=0)
    return pl.pallas_call(
        _net_body,
        grid=grid,
        in_specs=[
            row((_G_CONV, _NP * _NR)),
            row((_G_CONV * _NPG, _D_IN)),
            row((_G_CONV, _GLOB)),
            full2((2 * _D_IN, _D_H)),
            full2((1, _D_H)),
            full2((2 * _D_H, _D_O)),
            full2((1, _D_O)),
            full2((_GLOB, 8)),
            full2((1, 8)),
            full2((8, 8)),
            full2((1, 8)),
            full2((8, _GLOB)),
            full2((1, _GLOB)),
            pl.BlockSpec((_D_O, _NR, 2 * _D_H), lambda i: (0, 0, 0)),
            full2((_GLOB, _D_H)),
            full2((1, _D_H)),
            full2((_D_H, 1)),
            full2((1, 1)),
        ],
        out_specs=row((_G_CONV, 1)),
        out_shape=jax.ShapeDtypeStruct((_B, 1), jnp.float32),
        scratch_shapes=[
            pltpu.VMEM((_G_CONV * _NP, 2 * _D_IN), jnp.float32),
            pltpu.VMEM((_G_CONV * _NP, 2 * _D_H), jnp.float32),
        ],
        interpret=interpret,
    )(a, x, gf, w1cat, b_rel1.reshape(1, _D_H),
      w2cat, b_rel2.reshape(1, _D_O),
      wg1, bg1.reshape(1, 8), wg2, bg2.reshape(1, 8), wg3,
      bg3.reshape(1, _GLOB), w2e, w1g, bo1.reshape(1, _D_H), wo2,
      bo2.reshape(1, 1))


def _prep_head_weights(Wo1):
    w1r = Wo1[:_NPG * _D_O].reshape(_NPG, _D_O, _D_H)
    base = jnp.pad(w1r, ((0, _NP - _NPG), (0, 0), (0, 0)))
    base = base.transpose(1, 0, 2)
    zblk = jnp.zeros((_D_O, _NP, _D_H), jnp.float32)
    top = jnp.concatenate([base, zblk], axis=2)
    bot = jnp.concatenate([zblk, base], axis=2)
    return jnp.concatenate([top, bot], axis=1)


def kernel(x, edge_index, edge_attr, globalFeats, isTrain, W_rel1, b_rel1,
           W_root1, W_rel2, b_rel2, W_root2, Wg1, bg1, Wg2, bg2, Wg3, bg3,
           Wo1, bo1, Wo2, bo2):
    a = _build_adj(edge_index, edge_attr)
    w2e = _prep_head_weights(Wo1)
    w1g = Wo1[_NPG * _D_O:]
    return _net(a, x, globalFeats, W_rel1, b_rel1, W_root1,
                W_rel2, b_rel2, W_root2, Wg1, bg1, Wg2, bg2, Wg3, bg3,
                w2e, w1g, bo1, Wo2, bo2)


# speedup vs baseline: 92.1659x; 1.0013x over previous
import functools

import jax
import jax.numpy as jnp
from jax import lax
from jax.experimental import pallas as pl
from jax.experimental.pallas import tpu as pltpu
from jax.experimental.pallas import tpu_sc as plsc

_B = 1024
_NPG = 54
_NP = 64
_NR = 128
_EPG = 864
_D_IN = 64
_D_H = 128
_D_O = 4
_GLOB = 32

_G_CONV = 32


def _build_adj(edge_index, w):
    info = plsc.get_sparse_core_info()
    n_workers = info.num_cores * info.num_subcores
    gpw = _B // n_workers
    epw = gpw * _EPG
    mesh = plsc.VectorSubcoreMesh(core_axis_name="c", subcore_axis_name="s")

    @functools.partial(
        pl.kernel,
        out_type=jax.ShapeDtypeStruct((_B, _NP * _NR), jnp.float32),
        mesh=mesh,
        scratch_types=[
            pltpu.VMEM((epw,), jnp.int32),
            pltpu.VMEM((epw,), jnp.int32),
            pltpu.VMEM((epw,), jnp.float32),
            pltpu.VMEM((_NP * _NR,), jnp.float32),
            pltpu.VMEM((_NP * _NR,), jnp.float32),
            pltpu.SemaphoreType.DMA,
            pltpu.SemaphoreType.DMA,
            pltpu.SemaphoreType.DMA,
            pltpu.SemaphoreType.DMA,
        ],
        compiler_params=pltpu.CompilerParams(
            needs_layout_passes=False, skip_device_barrier=True),
        cost_estimate=pl.CostEstimate(
            flops=2_000_000, bytes_accessed=45_000_000, transcendentals=0),
    )
    def build(ei_hbm, w_hbm, a_hbm, src_v, dst_v, w_v, acc0, acc1,
              sem_s, sem_d, sem_w, sem_o):
        wid = lax.axis_index("s") * info.num_cores + lax.axis_index("c")
        ebase = wid * epw
        cs = pltpu.async_copy(ei_hbm.at[0, pl.ds(ebase, epw)], src_v, sem_s)
        cd = pltpu.async_copy(ei_hbm.at[1, pl.ds(ebase, epw)], dst_v, sem_d)
        cw = pltpu.async_copy(w_hbm.at[pl.ds(ebase, epw)], w_v, sem_w)

        def zero_all(j, c):
            for u in range(4):
                acc0[pl.ds(j * 64 + u * 16, 16)] = jnp.zeros(
                    (16,), jnp.float32)
                acc1[pl.ds(j * 64 + u * 16, 16)] = jnp.zeros(
                    (16,), jnp.float32)
            return c

        lax.fori_loop(0, _NP * _NR // 64, zero_all, 0)
        cs.wait()
        cd.wait()
        cw.wait()

        accs = (acc0, acc1)
        pending = [None, None]
        for gi in range(gpw):
            acc = accs[gi % 2]
            if pending[gi % 2] is not None:
                pending[gi % 2].wait()

            if gi >= 2:
                def zero_rows(d, c, acc=acc):
                    for u in range(4):
                        acc[pl.ds(d * _NR + u * 16, 16)] = jnp.zeros(
                            (16,), jnp.float32)
                    return c

                lax.fori_loop(0, _NPG, zero_rows, 0)

            g = wid * gpw + gi
            goff = g * _NPG
            e0 = gi * _EPG

            def edge48(i, c, acc=acc, e0=e0, goff=goff):
                for u in range(3):
                    o = e0 + i * 48 + u * 16
                    s = src_v[pl.ds(o, 16)]
                    d = dst_v[pl.ds(o, 16)]
                    ww = w_v[pl.ds(o, 16)]
                    idx = (d - goff) * _NR + (s - goff)
                    plsc.addupdate_scatter(acc, [idx], ww)
                return c

            lax.fori_loop(0, _EPG // 48, edge48, 0)
            pending[gi % 2] = pltpu.async_copy(acc, a_hbm.at[g], sem_o)
        for p in pending:
            if p is not None:
                p.wait()

    return build(edge_index, w)


def _net_body(a_ref, x_ref, gf_ref, w1cat_ref, br1_ref, w2cat_ref,
              br2_ref, wg1_ref, bg1_ref, wg2_ref, bg2_ref, wg3_ref,
              bg3_ref, w2e_ref, w1g_ref, bo1_ref, wo2_ref, bo2_ref, out_ref,
              cat1_s, cat2_s):
    pad_src = jnp.zeros((_NR - _NPG, _D_IN), jnp.float32)
    for r in range(_G_CONV):
        x_r = x_ref[pl.ds(r * _NPG, _NPG), :]
        xp128 = jnp.concatenate([x_r, pad_src], axis=0)
        a_r = jnp.reshape(a_ref[r], (_NP, _NR))
        cat1_s[pl.ds(r * _NP, _NP), :_D_IN] = jnp.dot(
            a_r, xp128, preferred_element_type=jnp.float32)
        cat1_s[pl.ds(r * _NP, _NP), _D_IN:] = xp128[:_NP]
    h1 = jnp.maximum(
        jnp.dot(cat1_s[...], w1cat_ref[...],
                preferred_element_type=jnp.float32) + br1_ref[...],
        0.0)
    cat2_s[:, _D_H:] = h1
    pad_h = jnp.zeros((_NR - _NP, _D_H), jnp.float32)
    for r in range(_G_CONV):
        a_r = jnp.reshape(a_ref[r], (_NP, _NR))
        h1p = jnp.concatenate(
            [cat2_s[pl.ds(r * _NP, _NP), _D_H:], pad_h], axis=0)
        cat2_s[pl.ds(r * _NP, _NP), :_D_H] = jnp.dot(
            a_r, h1p, preferred_element_type=jnp.float32)
    h2 = jnp.maximum(
        jnp.dot(cat2_s[...], w2cat_ref[...],
                preferred_element_type=jnp.float32) + br2_ref[...],
        0.0)
    row_ok = (lax.broadcasted_iota(jnp.int32, (_G_CONV * _NP, _D_O), 0)
              % _NP) < _NPG
    h2 = jnp.where(row_ok, h2, 0.0)

    h2t = jnp.transpose(h2, (1, 0))
    m = jnp.reshape(h2t, (_D_O, _G_CONV // 2, 2 * _NP))
    z2 = jnp.zeros((_G_CONV // 2, 2 * _D_H), jnp.float32)
    for c in range(_D_O):
        z2 = z2 + jnp.dot(m[c], w2e_ref[c],
                          preferred_element_type=jnp.float32)
    ze = jnp.reshape(z2, (_G_CONV, _D_H))

    gv = gf_ref[...]
    gv = jnp.maximum(jnp.dot(gv, wg1_ref[...],
                             preferred_element_type=jnp.float32)
                     + bg1_ref[...], 0.0)
    gv = jnp.maximum(jnp.dot(gv, wg2_ref[...],
                             preferred_element_type=jnp.float32)
                     + bg2_ref[...], 0.0)
    gv = jnp.maximum(jnp.dot(gv, wg3_ref[...],
                             preferred_element_type=jnp.float32)
                     + bg3_ref[...], 0.0)

    z = jnp.maximum(
        ze + jnp.dot(gv, w1g_ref[...], preferred_element_type=jnp.float32)
        + bo1_ref[...], 0.0)
    z = jnp.dot(z, wo2_ref[...], preferred_element_type=jnp.float32) \
        + bo2_ref[...]
    out_ref[...] = jax.nn.sigmoid(z)


def _net(a, x, gf, w_rel1, b_rel1, w_root1, w_rel2, b_rel2, w_root2,
         wg1, bg1, wg2, bg2, wg3, bg3, w2e, w1g, bo1, wo2, bo2,
         interpret=False):
    grid = (_B // _G_CONV,)
    row = lambda shape: pl.BlockSpec(shape, lambda i: (i, 0))
    full2 = lambda shape: pl.BlockSpec(shape, lambda i: (0, 0))
    w1cat = jnp.concatenate([w_rel1, w_root1], axis=0)
    w2cat = jnp.concatenate([w_rel2, w_root2], axis=0)
    return pl.pallas_call(
        _net_body,
        grid=grid,
        in_specs=[
            row((_G_CONV, _NP * _NR)),
            row((_G_CONV * _NPG, _D_IN)),
            row((_G_CONV, _GLOB)),
            full2((2 * _D_IN, _D_H)),
            full2((1, _D_H)),
            full2((2 * _D_H, _D_O)),
            full2((1, _D_O)),
            full2((_GLOB, 8)),
            full2((1, 8)),
            full2((8, 8)),
            full2((1, 8)),
            full2((8, _GLOB)),
            full2((1, _GLOB)),
            pl.BlockSpec((_D_O, _NR, 2 * _D_H), lambda i: (0, 0, 0)),
            full2((_GLOB, _D_H)),
            full2((1, _D_H)),
            full2((_D_H, 1)),
            full2((1, 1)),
        ],
        out_specs=row((_G_CONV, 1)),
        out_shape=jax.ShapeDtypeStruct((_B, 1), jnp.float32),
        scratch_shapes=[
            pltpu.VMEM((_G_CONV * _NP, 2 * _D_IN), jnp.float32),
            pltpu.VMEM((_G_CONV * _NP, 2 * _D_H), jnp.float32),
        ],
        interpret=interpret,
    )(a, x, gf, w1cat, b_rel1.reshape(1, _D_H),
      w2cat, b_rel2.reshape(1, _D_O),
      wg1, bg1.reshape(1, 8), wg2, bg2.reshape(1, 8), wg3,
      bg3.reshape(1, _GLOB), w2e, w1g, bo1.reshape(1, _D_H), wo2,
      bo2.reshape(1, 1))


def _prep_head_weights(Wo1):
    w1r = Wo1[:_NPG * _D_O].reshape(_NPG, _D_O, _D_H)
    base = jnp.pad(w1r, ((0, _NP - _NPG), (0, 0), (0, 0)))
    base = base.transpose(1, 0, 2)
    zblk = jnp.zeros((_D_O, _NP, _D_H), jnp.float32)
    top = jnp.concatenate([base, zblk], axis=2)
    bot = jnp.concatenate([zblk, base], axis=2)
    return jnp.concatenate([top, bot], axis=1)


def kernel(x, edge_index, edge_attr, globalFeats, isTrain, W_rel1, b_rel1,
           W_root1, W_rel2, b_rel2, W_root2, Wg1, bg1, Wg2, bg2, Wg3, bg3,
           Wo1, bo1, Wo2, bo2):
    a = _build_adj(edge_index, edge_attr)
    w2e = _prep_head_weights(Wo1)
    w1g = Wo1[_NPG * _D_O:]
    return _net(a, x, globalFeats, W_rel1, b_rel1, W_root1,
                W_rel2, b_rel2, W_root2, Wg1, bg1, Wg2, bg2, Wg3, bg3,
                w2e, w1g, bo1, Wo2, bo2)
